# baseline (speedup 1.0000x reference)
import functools

import jax
import jax.numpy as jnp
from jax import lax
from jax.experimental import pallas as pl
from jax.experimental.pallas import tpu as pltpu
from jax.experimental.pallas import tpu_sc as plsc

_BLK = 128


@functools.lru_cache(maxsize=None)
def _build(batch: int, hist: int, dim: int):
    info = plsc.get_sparse_core_info()
    nc, ns = info.num_cores, info.num_subcores
    nw = nc * ns
    assert batch % (nw * _BLK) == 0 and dim % 16 == 0
    b_per_w = batch // nw
    nbb = b_per_w // _BLK

    mesh = plsc.VectorSubcoreMesh(core_axis_name="c", subcore_axis_name="s")

    @functools.partial(
        pl.kernel,
        mesh=mesh,
        out_type=jax.ShapeDtypeStruct((hist, dim, batch), jnp.float32),
        scratch_types=[
            pltpu.VMEM((hist, b_per_w), jnp.int32),
            pltpu.VMEM((nbb, _BLK, dim), jnp.float32),
            pltpu.VMEM((nbb, dim, _BLK), jnp.float32),
        ]
        + [pltpu.SemaphoreType.DMA] * (2 * nbb),
        compiler_params=pltpu.CompilerParams(use_tc_tiling_on_sc=False, needs_layout_passes=False),
    )
    def gather_kernel(idxt_hbm, table_hbm, out_hbm, idx_v, bufs, obufs, *sems):
        in_sems, out_sems = sems[:nbb], sems[nbb:]
        wid = lax.axis_index("s") * nc + lax.axis_index("c")
        wb0 = wid * b_per_w

        pltpu.sync_copy(idxt_hbm.at[:, pl.ds(wb0, b_per_w)], idx_v)

        k_base = jnp.arange(16, dtype=jnp.int32)

        def fire(bb, h):
            pltpu.async_copy(
                table_hbm.at[idx_v.at[h, pl.ds(bb * _BLK, _BLK)]],
                bufs.at[bb],
                in_sems[bb],
            )

        def wait_gather(bb, h):
            pltpu.make_async_copy(
                table_hbm.at[idx_v.at[h, pl.ds(bb * _BLK, _BLK)]],
                bufs.at[bb],
                in_sems[bb],
            ).wait()

        def start_store(bb, h):
            pltpu.async_copy(
                obufs.at[bb],
                out_hbm.at[h, :, pl.ds(wb0 + bb * _BLK, _BLK)],
                out_sems[bb],
            )

        def wait_store(bb, h):
            pltpu.make_async_copy(
                obufs.at[bb],
                out_hbm.at[h, :, pl.ds(wb0 + bb * _BLK, _BLK)],
                out_sems[bb],
            ).wait()

        def transpose_unit(bb):
            buf = bufs.at[bb]
            obuf = obufs.at[bb]

            def c_chunk(cc, carry):
                for ci in range(8):
                    c = cc * 8 + ci
                    colv = jnp.broadcast_to(c, (16,)).astype(jnp.int32)
                    for kb in range(_BLK // 16):
                        v = plsc.load_gather(buf, [k_base + kb * 16, colv])
                        obuf[c, pl.ds(kb * 16, 16)] = v
                return carry

            lax.fori_loop(0, dim // 8, c_chunk, 0)

        for bb in range(nbb):
            fire(bb, 0)

        def h_body(h, carry):
            for bb in range(nbb):
                @pl.when(h > 0)
                def _():
                    wait_store(bb, h - 1)

                wait_gather(bb, h)
                transpose_unit(bb)

                @pl.when(h < hist - 1)
                def _():
                    fire(bb, h + 1)

                start_store(bb, h)
            return carry

        lax.fori_loop(0, hist, h_body, 0)

        for bb in range(nbb):
            wait_store(bb, hist - 1)

    return gather_kernel


def kernel(input, weight):
    batch, hist = input.shape
    dim = weight.shape[1]
    idxt = jnp.transpose(input)
    out = _build(batch, hist, dim)(idxt, weight)
    return jnp.transpose(out, (2, 0, 1))


# speedup vs baseline: 1.1493x; 1.1493x over previous
import functools

import jax
import jax.numpy as jnp
from jax import lax
from jax.experimental import pallas as pl
from jax.experimental.pallas import tpu as pltpu
from jax.experimental.pallas import tpu_sc as plsc

_BLK = 128


@functools.lru_cache(maxsize=None)
def _build(batch: int, hist: int, dim: int):
    info = plsc.get_sparse_core_info()
    nc, ns = info.num_cores, info.num_subcores
    nw = nc * ns
    assert batch % (nw * _BLK) == 0 and dim % 16 == 0
    b_per_w = batch // nw
    nbb = b_per_w // _BLK

    mesh = plsc.VectorSubcoreMesh(core_axis_name="c", subcore_axis_name="s")

    @functools.partial(
        pl.kernel,
        mesh=mesh,
        out_type=jax.ShapeDtypeStruct(
            (hist, dim // 8, batch // _BLK, 8, _BLK), jnp.float32),
        scratch_types=[
            pltpu.VMEM((hist, b_per_w), jnp.int32),
            pltpu.VMEM((nbb, _BLK, dim), jnp.float32),
            pltpu.VMEM((nbb, dim // 8, 8, _BLK), jnp.float32),
        ]
        + [pltpu.SemaphoreType.DMA] * (2 * nbb),
        compiler_params=pltpu.CompilerParams(
            use_tc_tiling_on_sc=False, needs_layout_passes=False),
    )
    def gather_kernel(idxt_hbm, table_hbm, out_hbm, idx_v, bufs, obufs, *sems):
        in_sems, out_sems = sems[:nbb], sems[nbb:]
        wid = lax.axis_index("s") * nc + lax.axis_index("c")
        wb0 = wid * b_per_w

        pltpu.sync_copy(idxt_hbm.at[:, pl.ds(wb0, b_per_w)], idx_v)

        k_iota = jnp.arange(16, dtype=jnp.int32)
        k_vecs = [k_iota + 16 * q for q in range(_BLK // 16)]

        def fire(bb, h):
            pltpu.async_copy(
                table_hbm.at[idx_v.at[h, pl.ds(bb * _BLK, _BLK)]],
                bufs.at[bb],
                in_sems[bb],
            )

        def wait_gather(bb, h):
            pltpu.make_async_copy(
                table_hbm.at[idx_v.at[h, pl.ds(bb * _BLK, _BLK)]],
                bufs.at[bb],
                in_sems[bb],
            ).wait()

        def start_store(bb, h):
            pltpu.async_copy(
                obufs.at[bb],
                out_hbm.at[h, :, wid * nbb + bb, :, :],
                out_sems[bb],
            )

        def wait_store(bb, h):
            pltpu.make_async_copy(
                obufs.at[bb],
                out_hbm.at[h, :, wid * nbb + bb, :, :],
                out_sems[bb],
            ).wait()

        def transpose_unit(bb):
            buf = bufs.at[bb]
            obuf = obufs.at[bb]

            def c_chunk(cc, carry):
                base = jnp.broadcast_to(cc * 8, (16,))
                for ci in range(8):
                    c_splat = base + ci
                    for q in range(_BLK // 16):
                        v = plsc.load_gather(buf, [k_vecs[q], c_splat])
                        obuf[cc, ci, pl.ds(16 * q, 16)] = v
                return carry

            lax.fori_loop(0, dim // 8, c_chunk, 0)

        for bb in range(nbb):
            fire(bb, 0)

        def h_body(h, carry):
            for bb in range(nbb):
                @pl.when(h > 0)
                def _():
                    wait_store(bb, h - 1)

                wait_gather(bb, h)
                transpose_unit(bb)

                @pl.when(h < hist - 1)
                def _():
                    fire(bb, h + 1)

                start_store(bb, h)
            return carry

        lax.fori_loop(0, hist, h_body, 0)

        for bb in range(nbb):
            wait_store(bb, hist - 1)

    return gather_kernel


def kernel(input, weight):
    batch, hist = input.shape
    dim = weight.shape[1]
    idxt = jnp.transpose(input)
    out5 = _build(batch, hist, dim)(idxt, weight)
    y = jnp.transpose(out5, (0, 1, 3, 2, 4)).reshape(hist, dim, batch)
    return jnp.transpose(y, (2, 0, 1))


# speedup vs baseline: 1.3208x; 1.1492x over previous
import functools

import jax
import jax.numpy as jnp
from jax import lax
from jax.experimental import pallas as pl
from jax.experimental.pallas import tpu as pltpu
from jax.experimental.pallas import tpu_sc as plsc

_BLK = 128


@functools.lru_cache(maxsize=None)
def _build(batch: int, hist: int, dim: int):
    info = plsc.get_sparse_core_info()
    nc, ns = info.num_cores, info.num_subcores
    nw = nc * ns
    assert batch % (nw * _BLK) == 0 and dim % 16 == 0
    b_per_w = batch // nw
    nbb = b_per_w // _BLK

    mesh = plsc.VectorSubcoreMesh(core_axis_name="c", subcore_axis_name="s")

    @functools.partial(
        pl.kernel,
        mesh=mesh,
        out_type=jax.ShapeDtypeStruct(
            (hist, dim // 8, batch // _BLK, 8, _BLK), jnp.float32),
        scratch_types=[
            pltpu.VMEM((hist, b_per_w), jnp.int32),
            pltpu.VMEM((nbb, _BLK, dim), jnp.float32),
            pltpu.VMEM((nbb, dim // 8, 8, _BLK), jnp.float32),
        ]
        + [pltpu.SemaphoreType.DMA] * (2 * nbb),
        compiler_params=pltpu.CompilerParams(
            use_tc_tiling_on_sc=False, needs_layout_passes=False),
    )
    def gather_kernel(idxt_hbm, table_hbm, out_hbm, idx_v, bufs, obufs, *sems):
        in_sems, out_sems = sems[:nbb], sems[nbb:]
        wid = lax.axis_index("s") * nc + lax.axis_index("c")
        wb0 = wid * b_per_w

        pltpu.sync_copy(idxt_hbm.at[:, pl.ds(wb0, b_per_w)], idx_v)

        k_iota = jnp.arange(16, dtype=jnp.int32)
        k_vecs = [k_iota + 16 * q for q in range(_BLK // 16)]

        def fire(bb, h):
            pltpu.async_copy(
                table_hbm.at[idx_v.at[h, pl.ds(bb * _BLK, _BLK)]],
                bufs.at[bb],
                in_sems[bb],
            )

        def wait_gather(bb, h):
            pltpu.make_async_copy(
                table_hbm.at[idx_v.at[h, pl.ds(bb * _BLK, _BLK)]],
                bufs.at[bb],
                in_sems[bb],
            ).wait()

        def start_store(bb, h):
            pltpu.async_copy(
                obufs.at[bb],
                out_hbm.at[h, :, wid * nbb + bb, :, :],
                out_sems[bb],
            )

        def wait_store(bb, h):
            pltpu.make_async_copy(
                obufs.at[bb],
                out_hbm.at[h, :, wid * nbb + bb, :, :],
                out_sems[bb],
            ).wait()

        def transpose_unit(bb):
            buf = bufs.at[bb]
            obuf = obufs.at[bb]

            def c_chunk(cc, carry):
                base = jnp.broadcast_to(cc * 8, (16,))
                for ci in range(8):
                    c_splat = base + ci
                    vs = [plsc.load_gather(buf, [k_vecs[q], c_splat])
                          for q in range(_BLK // 16)]
                    for q in range(_BLK // 16):
                        obuf[cc, ci, pl.ds(16 * q, 16)] = vs[q]
                return carry

            lax.fori_loop(0, dim // 8, c_chunk, 0)

        for bb in range(nbb):
            fire(bb, 0)

        def h_body(h, carry):
            for bb in range(nbb):
                @pl.when(h > 0)
                def _():
                    wait_store(bb, h - 1)

                wait_gather(bb, h)
                transpose_unit(bb)

                @pl.when(h < hist - 1)
                def _():
                    fire(bb, h + 1)

                start_store(bb, h)
            return carry

        lax.fori_loop(0, hist, h_body, 0)

        for bb in range(nbb):
            wait_store(bb, hist - 1)

    return gather_kernel


def kernel(input, weight):
    batch, hist = input.shape
    dim = weight.shape[1]
    idxt = jnp.transpose(input)
    out5 = _build(batch, hist, dim)(idxt, weight)
    y = jnp.transpose(out5, (0, 1, 3, 2, 4)).reshape(hist, dim, batch)
    return jnp.transpose(y, (2, 0, 1))


# speedup vs baseline: 2.3211x; 1.7573x over previous
import functools

import jax
import jax.numpy as jnp
from jax import lax
from jax.experimental import pallas as pl
from jax.experimental.pallas import tpu as pltpu
from jax.experimental.pallas import tpu_sc as plsc

_BLK = 128


@functools.lru_cache(maxsize=None)
def _build(batch: int, hist: int, dim: int):
    info = plsc.get_sparse_core_info()
    nc, ns = info.num_cores, info.num_subcores
    nw = nc * ns
    assert batch % (nw * _BLK) == 0 and dim % 16 == 0
    b_per_w = batch // nw
    nbb = b_per_w // _BLK

    mesh = plsc.VectorSubcoreMesh(core_axis_name="c", subcore_axis_name="s")

    @functools.partial(
        pl.kernel,
        mesh=mesh,
        out_type=jax.ShapeDtypeStruct(
            (hist, dim // 8, batch // _BLK, 8, _BLK), jnp.float32),
        scratch_types=[
            pltpu.VMEM((hist, nbb, _BLK), jnp.int32),
            pltpu.VMEM((nbb, _BLK, dim), jnp.float32),
            pltpu.VMEM((nbb, dim // 8, 8, _BLK + 1), jnp.float32),
        ]
        + [pltpu.SemaphoreType.DMA] * (2 * nbb),
        compiler_params=pltpu.CompilerParams(
            use_tc_tiling_on_sc=False, needs_layout_passes=False),
    )
    def gather_kernel(idxt_hbm, table_hbm, out_hbm, idx_v, bufs, obufs, *sems):
        in_sems, out_sems = sems[:nbb], sems[nbb:]
        wid = lax.axis_index("s") * nc + lax.axis_index("c")
        wb0 = wid * b_per_w

        pltpu.sync_copy(idxt_hbm.at[:, pl.ds(wid * nbb, nbb), :], idx_v)

        k_iota = jnp.arange(16, dtype=jnp.int32)
        cc_vecs = [(16 * q + k_iota) // 8 for q in range(dim // 16)]
        ci_vecs = [(16 * q + k_iota) % 8 for q in range(dim // 16)]

        def fire(bb, h):
            pltpu.async_copy(
                table_hbm.at[idx_v.at[h, bb]],
                bufs.at[bb],
                in_sems[bb],
            )

        def wait_gather(bb, h):
            pltpu.make_async_copy(
                table_hbm.at[idx_v.at[h, bb]],
                bufs.at[bb],
                in_sems[bb],
            ).wait()

        def start_store(bb, h):
            pltpu.async_copy(
                obufs.at[bb, :, :, pl.ds(0, _BLK)],
                out_hbm.at[h, :, wid * nbb + bb, :, :],
                out_sems[bb],
            )

        def wait_store(bb, h):
            pltpu.make_async_copy(
                obufs.at[bb, :, :, pl.ds(0, _BLK)],
                out_hbm.at[h, :, wid * nbb + bb, :, :],
                out_sems[bb],
            ).wait()

        def transpose_unit(bb):
            buf = bufs.at[bb]
            obuf = obufs.at[bb]

            def k_chunk(kb, carry):
                k0 = kb * 16
                for kk in range(16):
                    k = k0 + kk
                    k_splat = jnp.broadcast_to(k, (16,))
                    vs = [buf[k, pl.ds(16 * q, 16)]
                          for q in range(dim // 16)]
                    for q in range(dim // 16):
                        plsc.store_scatter(
                            obuf, [cc_vecs[q], ci_vecs[q], k_splat], vs[q])
                return carry

            lax.fori_loop(0, _BLK // 16, k_chunk, 0)

        for bb in range(nbb):
            fire(bb, 0)

        def h_body(h, carry):
            for bb in range(nbb):
                @pl.when(h > 0)
                def _():
                    wait_store(bb, h - 1)

                wait_gather(bb, h)
                transpose_unit(bb)

                @pl.when(h < hist - 1)
                def _():
                    fire(bb, h + 1)

                start_store(bb, h)
            return carry

        lax.fori_loop(0, hist, h_body, 0)

        for bb in range(nbb):
            wait_store(bb, hist - 1)

    return gather_kernel


def kernel(input, weight):
    batch, hist = input.shape
    dim = weight.shape[1]
    idxt = jnp.transpose(input).reshape(hist, batch // _BLK, _BLK)
    out5 = _build(batch, hist, dim)(idxt, weight)
    y = jnp.transpose(out5, (0, 1, 3, 2, 4)).reshape(hist, dim, batch)
    return jnp.transpose(y, (2, 0, 1))
